# baseline (speedup 1.0000x reference)
import functools

import jax
import jax.numpy as jnp
from jax import lax
from jax.experimental import pallas as pl
from jax.experimental.pallas import tpu as pltpu
from jax.experimental.pallas import tpu_sc as plsc

_C = 256
_HW = 180 * 180
_K = 6480
_BLK = 8192
_NB = (_HW + _BLK - 1) // _BLK

_NS = 16
_CHUNK = 2032
_PAD = _NS * _CHUNK
_NV = _CHUNK // 16

_ROUNDS = ((27, 4), (23, 4), (19, 4), (15, 4), (11, 4), (7, 4), (3, 4), (0, 3))


def _moments_body(bev_ref, prior_ref, score_ref, sim_ref):
    x = bev_ref[0]
    y = prior_ref[0]
    c = jnp.float32(_C)
    sx = jnp.sum(x, axis=0, keepdims=True)
    sy = jnp.sum(y, axis=0, keepdims=True)
    sxx = jnp.sum(x * x, axis=0, keepdims=True)
    syy = jnp.sum(y * y, axis=0, keepdims=True)
    sxy = jnp.sum(x * y, axis=0, keepdims=True)
    cxx = jnp.maximum(sxx - sx * sx / c, 0.0)
    cyy = jnp.maximum(syy - sy * sy / c, 0.0)
    cxy = sxy - sx * sy / c
    stdx = jnp.sqrt(cxx / (c - 1.0)) + 1e-6
    stdy = jnp.sqrt(cyy / (c - 1.0)) + 1e-6
    n1 = jnp.maximum(jnp.sqrt(cxx) / stdx, 1e-8)
    n2 = jnp.maximum(jnp.sqrt(cyy) / stdy, 1e-8)
    sim = (cxy / (stdx * stdy)) / (n1 * n2)
    score_ref[0] = sxx
    sim_ref[0] = sim


def _sc_select_body(keys_hbm, sim_hbm, out_hbm,
                    sim_v, keys_v, hist_v, stage_v, out_v,
                    shared_i, shared_f, loc_i, loc_f):
    c = lax.axis_index("c")
    s = lax.axis_index("s")
    lanes = lax.iota(jnp.int32, 16)
    ones = jnp.ones((16,), jnp.int32)

    for bb in range(2):
        b = c * 2 + bb
        pltpu.sync_copy(keys_hbm.at[b, s], keys_v)
        pltpu.sync_copy(sim_hbm.at[b, s], sim_v)

        p = jnp.zeros((16,), jnp.int32)
        n_above = jnp.zeros((16,), jnp.int32)

        for bpos, fbits in _ROUNDS:
            nf = 1 << fbits
            hist_v[...] = jnp.zeros((16,), jnp.int32)
            pref = lax.shift_right_arithmetic(p, bpos + fbits)

            def hist_body(v, carry, _bpos=bpos, _fbits=fbits, _pref=pref):
                kv = keys_v[pl.ds(v * 16, 16)]
                f = lax.shift_right_arithmetic(kv, _bpos) & ((1 << _fbits) - 1)
                match = lax.shift_right_arithmetic(kv, _bpos + _fbits) == _pref
                plsc.addupdate_scatter(hist_v, [f], ones, mask=match)
                return carry

            lax.fori_loop(0, _NV, hist_body, jnp.int32(0))

            pltpu.sync_copy(hist_v, shared_i.at[s])
            plsc.subcore_barrier()
            pltpu.sync_copy(shared_i, loc_i)
            plsc.subcore_barrier()
            tot = jnp.zeros((16,), jnp.int32)
            for r in range(_NS):
                tot = tot + loc_i[r]
            sfx = lax.rev(jnp.cumsum(lax.rev(tot, (0,))), (0,))
            ge = ((n_above + sfx) >= _K) & (lanes < nf)
            t = plsc.all_reduce_population_count(ge) - 1
            stp1 = jnp.sum(jnp.where(lanes == t + 1, sfx, 0))
            n_above = n_above + stp1
            p = p | lax.shift_left(t, bpos)

        def stats_body(v, carry):
            g_n, g_s, e_n, e_s = carry
            sl = pl.ds(v * 16, 16)
            kv = keys_v[sl]
            sv = sim_v[sl]
            gt = kv > p
            eq = kv == p
            zf = jnp.zeros((16,), jnp.float32)
            g_n = g_n + jnp.where(gt, 1.0, 0.0)
            g_s = g_s + jnp.where(gt, sv, zf)
            e_n = e_n + jnp.where(eq, 1.0, 0.0)
            e_s = e_s + jnp.where(eq, sv, zf)
            return g_n, g_s, e_n, e_s

        z = jnp.zeros((16,), jnp.float32)
        g_n, g_s, e_n, e_s = lax.fori_loop(0, _NV, stats_body, (z, z, z, z))
        packed = (jnp.where(lanes == 0, jnp.sum(g_n), 0.0)
                  + jnp.where(lanes == 1, jnp.sum(g_s), 0.0)
                  + jnp.where(lanes == 2, jnp.sum(e_n), 0.0)
                  + jnp.where(lanes == 3, jnp.sum(e_s), 0.0))
        stage_v[...] = packed
        pltpu.sync_copy(stage_v, shared_f.at[s])
        plsc.subcore_barrier()
        pltpu.sync_copy(shared_f, loc_f)
        plsc.subcore_barrier()
        totf = jnp.zeros((16,), jnp.float32)
        for r in range(_NS):
            totf = totf + loc_f[r]
        n_gt = jnp.sum(jnp.where(lanes == 0, totf, 0.0))
        s_gt = jnp.sum(jnp.where(lanes == 1, totf, 0.0))
        n_eq = jnp.sum(jnp.where(lanes == 2, totf, 0.0))
        s_eq = jnp.sum(jnp.where(lanes == 3, totf, 0.0))
        n_gt_v = jnp.full((16,), n_gt, jnp.float32)
        s_gt_v = jnp.full((16,), s_gt, jnp.float32)
        n_eq_v = jnp.full((16,), n_eq, jnp.float32)
        s_eq_v = jnp.full((16,), s_eq, jnp.float32)
        frac_v = (jnp.float32(_K) - n_gt_v) / jnp.maximum(n_eq_v, 1.0)
        total_v = s_gt_v + frac_v * s_eq_v

        @pl.when(s == 0)
        def _write():
            out_v[...] = total_v
            pltpu.sync_copy(out_v, out_hbm.at[b])


def kernel(bev_map, prior_warp, dx_m, dy_m, dtheta):
    B, C, H, W = bev_map.shape
    bev = bev_map.reshape(B, C, H * W)
    prior = prior_warp.reshape(B, C, H * W)

    score, sim = pl.pallas_call(
        _moments_body,
        grid=(B, _NB),
        in_specs=[
            pl.BlockSpec((1, C, _BLK), lambda b, j: (b, 0, j)),
            pl.BlockSpec((1, C, _BLK), lambda b, j: (b, 0, j)),
        ],
        out_specs=[
            pl.BlockSpec((1, 1, _BLK), lambda b, j: (b, 0, j)),
            pl.BlockSpec((1, 1, _BLK), lambda b, j: (b, 0, j)),
        ],
        out_shape=[
            jax.ShapeDtypeStruct((B, 1, H * W), jnp.float32),
            jax.ShapeDtypeStruct((B, 1, H * W), jnp.float32),
        ],
    )(bev, prior)

    keys3 = jax.lax.bitcast_convert_type(
        jnp.pad(score[:, 0, :], ((0, 0), (0, _PAD - H * W)),
                constant_values=-1.0), jnp.int32).reshape(B, _NS, _CHUNK)
    sim3 = jnp.pad(sim[:, 0, :], ((0, 0), (0, _PAD - H * W))
                   ).reshape(B, _NS, _CHUNK)

    mesh = plsc.VectorSubcoreMesh(core_axis_name="c", subcore_axis_name="s")
    sc_select = functools.partial(
        pl.kernel,
        out_type=jax.ShapeDtypeStruct((B, 16), jnp.float32),
        mesh=mesh,
        compiler_params=pltpu.CompilerParams(needs_layout_passes=False),
        scratch_types=[
            pltpu.VMEM((_CHUNK,), jnp.float32),
            pltpu.VMEM((_CHUNK,), jnp.int32),
            pltpu.VMEM((16,), jnp.int32),
            pltpu.VMEM((16,), jnp.float32),
            pltpu.VMEM((16,), jnp.float32),
            pltpu.VMEM_SHARED((_NS, 16), jnp.int32),
            pltpu.VMEM_SHARED((_NS, 16), jnp.float32),
            pltpu.VMEM((_NS, 16), jnp.int32),
            pltpu.VMEM((_NS, 16), jnp.float32),
        ],
    )(_sc_select_body)

    sums = sc_select(keys3, sim3)

    align_loss = 1.0 - jnp.sum(sums[:, 0]) / jnp.float32(B * _K)
    reg_loss = jnp.mean(dx_m ** 2 + dy_m ** 2) + jnp.mean(dtheta ** 2)
    return align_loss + 0.1 * reg_loss


# speedup vs baseline: 1.0402x; 1.0402x over previous
import jax
import jax.numpy as jnp
from jax.experimental import pallas as pl
from jax.experimental.pallas import tpu as pltpu

_C = 256
_HW = 180 * 180
_K = 6480
_BLK = 8192
_NB = (_HW + _BLK - 1) // _BLK


def _fused_body(bev_ref, prior_ref, out_ref, score_scr, sim_scr):
    j = pl.program_id(1)
    x = bev_ref[0]
    y = prior_ref[0]
    c = jnp.float32(_C)
    sx = jnp.sum(x, axis=0, keepdims=True)
    sy = jnp.sum(y, axis=0, keepdims=True)
    sxx = jnp.sum(x * x, axis=0, keepdims=True)
    syy = jnp.sum(y * y, axis=0, keepdims=True)
    sxy = jnp.sum(x * y, axis=0, keepdims=True)
    cxx = jnp.maximum(sxx - sx * sx / c, 0.0)
    cyy = jnp.maximum(syy - sy * sy / c, 0.0)
    cxy = sxy - sx * sy / c
    stdx = jnp.sqrt(cxx / (c - 1.0)) + 1e-6
    stdy = jnp.sqrt(cyy / (c - 1.0)) + 1e-6
    n1 = jnp.maximum(jnp.sqrt(cxx) / stdx, 1e-8)
    n2 = jnp.maximum(jnp.sqrt(cyy) / stdy, 1e-8)
    sim = (cxy / (stdx * stdy)) / (n1 * n2)
    score_scr[pl.ds(j, 1), :] = sxx
    sim_scr[pl.ds(j, 1), :] = sim

    @pl.when(j == _NB - 1)
    def _select():
        score = score_scr[...]
        simv = sim_scr[...]
        idx = (jax.lax.broadcasted_iota(jnp.int32, score.shape, 0) * _BLK
               + jax.lax.broadcasted_iota(jnp.int32, score.shape, 1))
        valid = idx < _HW
        keys = jnp.where(valid, jax.lax.bitcast_convert_type(score, jnp.int32),
                         jnp.int32(-1))

        def radix_select(karr, kk, blist):
            p = jnp.int32(0)
            for b in blist:
                t = jnp.int32(0)
                for m in range(1, 8):
                    cnt = jnp.sum((karr >= (p | (m << b))).astype(jnp.int32))
                    t = t + (cnt >= kk).astype(jnp.int32)
                p = p | jnp.left_shift(t, b)
            cnt = jnp.sum((karr >= (p | 1)).astype(jnp.int32))
            return p | (cnt >= kk).astype(jnp.int32)

        v = radix_select(keys, _K, (28, 25, 22, 19, 16, 13, 10, 7, 4, 1))
        n_gt = jnp.sum((keys > v).astype(jnp.int32))
        need = _K - n_gt
        eq = (keys == v) & valid
        key2 = jnp.where(eq, 32767 - idx, -1)
        v2 = radix_select(key2, need, (12, 9, 6, 3, 0))
        sel = (keys > v) | (key2 >= v2)
        out_ref[...] = jnp.sum(jnp.where(sel, simv, 0.0)).reshape(1, 1, 1)


def kernel(bev_map, prior_warp, dx_m, dy_m, dtheta):
    B, C, H, W = bev_map.shape
    bev = bev_map.reshape(B, C, H * W)
    prior = prior_warp.reshape(B, C, H * W)

    sums = pl.pallas_call(
        _fused_body,
        grid=(B, _NB),
        in_specs=[
            pl.BlockSpec((1, C, _BLK), lambda b, j: (b, 0, j)),
            pl.BlockSpec((1, C, _BLK), lambda b, j: (b, 0, j)),
        ],
        out_specs=pl.BlockSpec((1, 1, 1), lambda b, j: (b, 0, 0)),
        out_shape=jax.ShapeDtypeStruct((B, 1, 1), jnp.float32),
        scratch_shapes=[
            pltpu.VMEM((_NB, _BLK), jnp.float32),
            pltpu.VMEM((_NB, _BLK), jnp.float32),
        ],
    )(bev, prior)

    align_loss = 1.0 - jnp.sum(sums) / jnp.float32(B * _K)
    reg_loss = jnp.mean(dx_m ** 2 + dy_m ** 2) + jnp.mean(dtheta ** 2)
    return align_loss + 0.1 * reg_loss
